# baseline (speedup 1.0000x reference)
import functools

import jax
import jax.numpy as jnp
from jax import lax
from jax.experimental import pallas as pl
from jax.experimental.pallas import tpu as pltpu
from jax.experimental.pallas import tpu_sc as plsc


def _mlp_kernel(x_ref, W1_ref, b1_ref, W2_ref, b2_ref,
                Ws1_ref, bs1_ref, Ws2_ref, bs2_ref, x1_ref, xs_ref):
    x = x_ref[...]
    h1 = jax.nn.relu(jnp.dot(x, W1_ref[...], preferred_element_type=jnp.float32)
                     + b1_ref[...])
    x1_ref[...] = jax.nn.relu(
        jnp.dot(h1, W2_ref[...], preferred_element_type=jnp.float32) + b2_ref[...])
    hs = jax.nn.relu(jnp.dot(x, Ws1_ref[...], preferred_element_type=jnp.float32)
                     + bs1_ref[...])
    xs_ref[...] = jax.nn.relu(
        jnp.dot(hs, Ws2_ref[...], preferred_element_type=jnp.float32) + bs2_ref[...])


_R = 8
_JC = 256


def _sc_msg_kernel(A_hbm, W_hbm, x1_hbm, xs_hbm, out_hbm,
                   x1v, wv, av, xsv, outv, *, n, f, n_workers):
    rows_per_worker = (2 * n) // n_workers
    c = lax.axis_index("c")
    s = lax.axis_index("s")
    wid = c * 16 + s
    per_batch = n_workers // 2
    batch = wid // per_batch
    i0 = (wid % per_batch) * rows_per_worker

    pltpu.sync_copy(x1_hbm.at[batch], x1v)

    def group(g, _):
        ib = i0 + g * _R
        pltpu.sync_copy(A_hbm.at[batch, pl.ds(ib, _R), :], av)
        pltpu.sync_copy(xs_hbm.at[batch, pl.ds(ib, _R), :], xsv)

        def chunk(jc, accs):
            for r in range(_R):
                pltpu.sync_copy(
                    W_hbm.at[batch, ib + r, pl.ds(jc * _JC * f, _JC * f)],
                    wv.at[r])

            def body(jb, accs):
                jj = jc * _JC + jb * 16
                a16 = [av[r, pl.ds(jj, 16)] for r in range(_R)]
                accs = list(accs)
                for l in range(16):
                    x116 = x1v[pl.ds((jj + l) * f, f)]
                    for r in range(_R):
                        w16 = wv[r, pl.ds((jb * 16 + l) * f, f)]
                        accs[r] = accs[r] + (w16 * x116) * a16[r][l]
                return tuple(accs)

            return lax.fori_loop(0, _JC // 16, body, accs)

        accs = lax.fori_loop(
            0, n // _JC, chunk,
            tuple(jnp.zeros((f,), jnp.float32) for _ in range(_R)))

        ones = jnp.ones((f,), jnp.float32)
        iota = lax.iota(jnp.int32, f)
        for r in range(_R):
            def sbody(k, sacc, r=r):
                return sacc + av[r, pl.ds(k * f, f)]
            sv = lax.fori_loop(0, n // f, sbody, jnp.zeros((f,), jnp.float32))
            for k in (8, 4, 2, 1):
                sv = sv + jnp.take(sv, iota ^ k)
            scale16 = jnp.where(sv != 0.0, ones / sv, 0.0)
            outv[r, :] = accs[r] * scale16 + xsv[r, :]
        pltpu.sync_copy(outv, out_hbm.at[batch, pl.ds(ib, _R), :])
        return 0

    lax.fori_loop(0, rows_per_worker // _R, group, 0)


@jax.jit
def kernel(A, W, x, W1, b1, W2, b2, Ws1, bs1, Ws2, bs2):
    b, n, din = x.shape
    f = W.shape[-1]

    x2d = x.reshape(b * n, din)
    x1f, xsf = pl.pallas_call(
        _mlp_kernel,
        out_shape=(
            jax.ShapeDtypeStruct((b * n, f), jnp.float32),
            jax.ShapeDtypeStruct((b * n, f), jnp.float32),
        ),
    )(x2d, W1, b1.reshape(1, f), W2, b2.reshape(1, f),
      Ws1, bs1.reshape(1, f), Ws2, bs2.reshape(1, f))
    x1 = x1f.reshape(b, n * f)
    xs = xsf.reshape(b, n, f)

    Wlin = W.reshape(b, n, n * f)
    n_workers = 32
    mesh = plsc.VectorSubcoreMesh(core_axis_name="c", subcore_axis_name="s")
    sc = functools.partial(
        pl.kernel,
        mesh=mesh,
        out_type=jax.ShapeDtypeStruct((b, n, f), jnp.float32),
        scratch_types=[
            pltpu.VMEM((n * f,), jnp.float32),
            pltpu.VMEM((_R, _JC * f), jnp.float32),
            pltpu.VMEM((_R, n), jnp.float32),
            pltpu.VMEM((_R, f), jnp.float32),
            pltpu.VMEM((_R, f), jnp.float32),
        ],
    )(functools.partial(_sc_msg_kernel, n=n, f=f, n_workers=n_workers))
    x2 = sc(A, Wlin, x1, xs)

    return (W, x2)
